# baseline (speedup 1.0000x reference)
import functools

import jax
import jax.numpy as jnp
from jax.experimental import pallas as pl
from jax.experimental.pallas import tpu as pltpu

_ANCHORS = ((116.0, 90.0), (156.0, 198.0), (373.0, 326.0))
_NUM_CLASSES = 80
_IMG_SIZE = 512.0
_IGNORE_THRESH = 0.5
_EPS = 1e-06


def _softplus_neg_abs(x):
    return jnp.log(1.0 + jnp.exp(-jnp.abs(x)))


def _to_rl(v, sel_even, sel_odd):
    e = jax.lax.dot(sel_even, v, preferred_element_type=jnp.float32)
    o = jax.lax.dot(sel_odd, v, preferred_element_type=jnp.float32)
    return jnp.concatenate([e, o], axis=1)


def _loss_kernel(plow_ref, cls_ref, t0_ref, t1_ref, t2_ref, t3_ref, t4_ref,
                 t5_ref, out_ref, acc_ref, *, h, w, nb, anchors_grid):
    A = len(anchors_grid)
    C = _NUM_CLASSES
    f32 = jnp.float32
    R, L = 32, 128

    @pl.when(pl.program_id(0) == 0)
    def _init():
        for j in range(6):
            acc_ref[j] = f32(0.0)

    p = plow_ref[0]
    x_off = jax.lax.broadcasted_iota(jnp.int32, (h, w), 0).astype(f32)
    y_off = jax.lax.broadcasted_iota(jnp.int32, (h, w), 1).astype(f32)
    cidx = jax.lax.broadcasted_iota(jnp.int32, (C, R, L), 0)
    rr = jax.lax.broadcasted_iota(jnp.int32, (R, h), 0)
    ss = jax.lax.broadcasted_iota(jnp.int32, (R, h), 1)
    sel_even = (ss == 2 * rr).astype(f32)
    sel_odd = (ss == 2 * rr + 1).astype(f32)

    s_obj_bce = f32(0.0)
    s_all_bce = f32(0.0)
    n_obj = f32(0.0)
    s_box = f32(0.0)
    s_cls = f32(0.0)
    n_tgt = f32(0.0)

    for a in range(A):
        aw, ah = anchors_grid[a]
        px = p[4 * a + 0]
        py = p[4 * a + 1]
        pw = p[4 * a + 2]
        ph = p[4 * a + 3]
        obj = p[4 * A + a]

        tx = (t0_ref[0, a] * w - x_off) * (1.0 / aw)
        ty = (t1_ref[0, a] * h - y_off) * (1.0 / ah)
        tw = (t2_ref[0, a] * w - x_off) * (1.0 / aw)
        th = (t3_ref[0, a] * h - y_off) * (1.0 / ah)
        tgt_obj = t4_ref[0, a]
        tgt_cls = t5_ref[0, a]

        ax1 = px - pw * 0.5
        ax2 = px + pw * 0.5
        ay1 = py - ph * 0.5
        ay2 = py + ph * 0.5
        bx1 = tx - tw * 0.5
        bx2 = tx + tw * 0.5
        by1 = ty - th * 0.5
        by2 = ty + th * 0.5
        iw = jnp.clip(jnp.minimum(ax2, bx2) - jnp.maximum(ax1, bx1), 0.0)
        ih = jnp.clip(jnp.minimum(ay2, by2) - jnp.maximum(ay1, by1), 0.0)
        inter = iw * ih
        area_a = jnp.clip(ax2 - ax1, 0.0) * jnp.clip(ay2 - ay1, 0.0)
        area_b = jnp.clip(bx2 - bx1, 0.0) * jnp.clip(by2 - by1, 0.0)
        iou = inter / (area_a + area_b - inter + 1e-09)

        tgt_mask = tgt_obj > 0.0
        obj_mask = jnp.logical_and(iou > _IGNORE_THRESH, tgt_mask)
        m = obj_mask.astype(f32)

        obj_bce = jnp.maximum(obj, 0.0) - obj * tgt_obj + _softplus_neg_abs(obj)
        s_all_bce += jnp.sum(obj_bce)
        s_obj_bce += jnp.sum(obj_bce * m)
        n_obj += jnp.sum(m)
        n_tgt += jnp.sum(tgt_mask.astype(f32))

        box_mse = ((px - tx) ** 2 + (py - ty) ** 2
                   + (pw - tw) ** 2 + (ph - th) ** 2) * 0.25
        s_box += jnp.sum(box_mse * m)

        cls = cls_ref[0, 5 * A + a * C:5 * A + (a + 1) * C].astype(f32)
        m_rl = _to_rl(m, sel_even, sel_odd)
        k_rl = _to_rl(tgt_cls, sel_even, sel_odd).astype(jnp.int32)
        sp = jnp.maximum(cls, 0.0) + _softplus_neg_abs(cls)
        q = sp - jnp.where(cidx == k_rl[None], cls, 0.0)
        cls_bce = jnp.sum(q, axis=0) * (1.0 / C)
        s_cls += jnp.sum(cls_bce * m_rl)

    acc_ref[0] += s_obj_bce
    acc_ref[1] += s_all_bce
    acc_ref[2] += n_obj
    acc_ref[3] += s_box
    acc_ref[4] += s_cls
    acc_ref[5] += n_tgt

    @pl.when(pl.program_id(0) == nb - 1)
    def _finalize():
        so = acc_ref[0]
        sa = acc_ref[1]
        no = acc_ref[2]
        sb = acc_ref[3]
        sc = acc_ref[4]
        nt = acc_ref[5]
        total = f32(nb * A * h * w)
        n_noobj = total - no
        s_noobj = sa - so
        obj_loss = (total / (no + _EPS)) * (so / (no + _EPS))
        noobj_loss = (total / (n_noobj + _EPS)) * (s_noobj / (n_noobj + _EPS))
        box_loss = sb / (no + _EPS)
        class_loss = (total / (nt + _EPS)) * (sc / (no + _EPS))
        out_ref[0, 0] = obj_loss + noobj_loss + box_loss + class_loss


def kernel(predictions, targets):
    b, ch, h, w = predictions.shape
    A = len(_ANCHORS)
    C = _NUM_CLASSES
    stride = _IMG_SIZE / h
    anchors_grid = tuple((aw / stride, ah / stride) for aw, ah in _ANCHORS)
    R, L = 32, 128
    assert h * w == R * L and ch == A * (5 + C)
    nlow = 5 * A

    pred_bf16 = predictions.reshape(b, ch, R, L).astype(jnp.bfloat16)
    tplanes = [targets[..., j] for j in range(6)]

    plane_spec = pl.BlockSpec((1, A, h, w), lambda i: (i, 0, 0, 0))
    loss = pl.pallas_call(
        functools.partial(_loss_kernel, h=h, w=w, nb=b,
                          anchors_grid=anchors_grid),
        grid=(b,),
        in_specs=[
            pl.BlockSpec((1, nlow, h, w), lambda i: (i, 0, 0, 0)),
            pl.BlockSpec((1, ch, R, L), lambda i: (i, 0, 0, 0)),
        ] + [plane_spec] * 6,
        out_specs=pl.BlockSpec(memory_space=pltpu.SMEM),
        out_shape=jax.ShapeDtypeStruct((1, 1), jnp.float32),
        scratch_shapes=[pltpu.SMEM((6,), jnp.float32)],
    )(predictions, pred_bf16, *tplanes)

    return loss[0, 0]


# speedup vs baseline: 1.8379x; 1.8379x over previous
import functools

import jax
import jax.numpy as jnp
from jax.experimental import pallas as pl
from jax.experimental.pallas import tpu as pltpu

_ANCHORS = ((116.0, 90.0), (156.0, 198.0), (373.0, 326.0))
_NUM_CLASSES = 80
_IMG_SIZE = 512.0
_IGNORE_THRESH = 0.5
_EPS = 1e-06


def _softplus_neg_abs(x):
    return jnp.log(1.0 + jnp.exp(-jnp.abs(x)))


def _loss_kernel(pred_ref, t0_ref, t1_ref, t2_ref, t3_ref, t4_ref, t5_ref,
                 out_ref, acc_ref, *, h, w, nb, anchors_grid):
    A = len(anchors_grid)
    C = _NUM_CLASSES
    f32 = jnp.float32
    R, L = 16, 128

    @pl.when(jnp.logical_and(pl.program_id(0) == 0, pl.program_id(1) == 0))
    def _init():
        for j in range(6):
            acc_ref[j] = f32(0.0)

    p = pred_ref[0]
    lin = (jax.lax.broadcasted_iota(jnp.int32, (R, L), 0) * L
           + jax.lax.broadcasted_iota(jnp.int32, (R, L), 1)
           + pl.program_id(1) * (R * L))
    x_off = (lin // w).astype(f32)
    y_off = (lin % w).astype(f32)
    cidx = jax.lax.broadcasted_iota(jnp.int32, (C, R, L), 0)

    s_obj_bce = f32(0.0)
    s_all_bce = f32(0.0)
    n_obj = f32(0.0)
    s_box = f32(0.0)
    s_cls = f32(0.0)
    n_tgt = f32(0.0)

    for a in range(A):
        aw, ah = anchors_grid[a]
        px = p[4 * a + 0]
        py = p[4 * a + 1]
        pw = p[4 * a + 2]
        ph = p[4 * a + 3]
        obj = p[4 * A + a]
        cls = p[5 * A + a * C:5 * A + (a + 1) * C]

        tx = (t0_ref[0, a] * w - x_off) * (1.0 / aw)
        ty = (t1_ref[0, a] * h - y_off) * (1.0 / ah)
        tw = (t2_ref[0, a] * w - x_off) * (1.0 / aw)
        th = (t3_ref[0, a] * h - y_off) * (1.0 / ah)
        tgt_obj = t4_ref[0, a]
        tgt_cls = t5_ref[0, a]

        ax1 = px - pw * 0.5
        ax2 = px + pw * 0.5
        ay1 = py - ph * 0.5
        ay2 = py + ph * 0.5
        bx1 = tx - tw * 0.5
        bx2 = tx + tw * 0.5
        by1 = ty - th * 0.5
        by2 = ty + th * 0.5
        iw = jnp.clip(jnp.minimum(ax2, bx2) - jnp.maximum(ax1, bx1), 0.0)
        ih = jnp.clip(jnp.minimum(ay2, by2) - jnp.maximum(ay1, by1), 0.0)
        inter = iw * ih
        area_a = jnp.clip(ax2 - ax1, 0.0) * jnp.clip(ay2 - ay1, 0.0)
        area_b = jnp.clip(bx2 - bx1, 0.0) * jnp.clip(by2 - by1, 0.0)
        iou = inter / (area_a + area_b - inter + 1e-09)

        tgt_mask = tgt_obj > 0.0
        obj_mask = jnp.logical_and(iou > _IGNORE_THRESH, tgt_mask)
        m = obj_mask.astype(f32)

        obj_bce = jnp.maximum(obj, 0.0) - obj * tgt_obj + _softplus_neg_abs(obj)
        s_all_bce += jnp.sum(obj_bce)
        s_obj_bce += jnp.sum(obj_bce * m)
        n_obj += jnp.sum(m)
        n_tgt += jnp.sum(tgt_mask.astype(f32))

        box_mse = ((px - tx) ** 2 + (py - ty) ** 2
                   + (pw - tw) ** 2 + (ph - th) ** 2) * 0.25
        s_box += jnp.sum(box_mse * m)

        sp = jnp.maximum(cls, 0.0) + _softplus_neg_abs(cls)
        q = sp - jnp.where(cidx == tgt_cls[None].astype(jnp.int32), cls, 0.0)
        cls_bce = jnp.sum(q, axis=0) * (1.0 / C)
        s_cls += jnp.sum(cls_bce * m)

    acc_ref[0] += s_obj_bce
    acc_ref[1] += s_all_bce
    acc_ref[2] += n_obj
    acc_ref[3] += s_box
    acc_ref[4] += s_cls
    acc_ref[5] += n_tgt

    @pl.when(jnp.logical_and(pl.program_id(0) == nb - 1, pl.program_id(1) == 1))
    def _finalize():
        so = acc_ref[0]
        sa = acc_ref[1]
        no = acc_ref[2]
        sb = acc_ref[3]
        sc = acc_ref[4]
        nt = acc_ref[5]
        total = f32(nb * A * h * w)
        n_noobj = total - no
        s_noobj = sa - so
        obj_loss = (total / (no + _EPS)) * (so / (no + _EPS))
        noobj_loss = (total / (n_noobj + _EPS)) * (s_noobj / (n_noobj + _EPS))
        box_loss = sb / (no + _EPS)
        class_loss = (total / (nt + _EPS)) * (sc / (no + _EPS))
        out_ref[0, 0] = obj_loss + noobj_loss + box_loss + class_loss


def kernel(predictions, targets):
    b, ch, h, w = predictions.shape
    A = len(_ANCHORS)
    stride = _IMG_SIZE / h
    anchors_grid = tuple((aw / stride, ah / stride) for aw, ah in _ANCHORS)
    R, L = 16, 128
    assert h * w == 2 * R * L

    pred = predictions.reshape(b, ch, 2 * R, L)
    tplanes = [targets[..., j].reshape(b, A, 2 * R, L) for j in range(6)]

    plane_spec = pl.BlockSpec((1, A, R, L), lambda i, k: (i, 0, k, 0))
    loss = pl.pallas_call(
        functools.partial(_loss_kernel, h=h, w=w, nb=b,
                          anchors_grid=anchors_grid),
        grid=(b, 2),
        in_specs=[pl.BlockSpec((1, ch, R, L), lambda i, k: (i, 0, k, 0))]
        + [plane_spec] * 6,
        out_specs=pl.BlockSpec(memory_space=pltpu.SMEM),
        out_shape=jax.ShapeDtypeStruct((1, 1), jnp.float32),
        scratch_shapes=[pltpu.SMEM((6,), jnp.float32)],
    )(pred, *tplanes)

    return loss[0, 0]


# speedup vs baseline: 1.8646x; 1.0145x over previous
import functools

import jax
import jax.numpy as jnp
from jax.experimental import pallas as pl
from jax.experimental.pallas import tpu as pltpu

_ANCHORS = ((116.0, 90.0), (156.0, 198.0), (373.0, 326.0))
_NUM_CLASSES = 80
_IMG_SIZE = 512.0
_IGNORE_THRESH = 0.5
_EPS = 1e-06


def _softplus_neg_abs(x):
    return jnp.log(1.0 + jnp.exp(-jnp.abs(x)))


def _loss_kernel(pred_ref, t0_ref, t1_ref, t2_ref, t3_ref, t4_ref, t5_ref,
                 out_ref, acc_ref, *, h, w, nb, anchors_grid):
    A = len(anchors_grid)
    C = _NUM_CLASSES
    f32 = jnp.float32
    R, L = 32, 128

    @pl.when(pl.program_id(0) == 0)
    def _init():
        for j in range(6):
            acc_ref[j] = f32(0.0)

    p = pred_ref[0]
    lin = (jax.lax.broadcasted_iota(jnp.int32, (R, L), 0) * L
           + jax.lax.broadcasted_iota(jnp.int32, (R, L), 1))
    x_off = (lin // w).astype(f32)
    y_off = (lin % w).astype(f32)
    cidx = jax.lax.broadcasted_iota(jnp.int32, (C, R, L), 0)

    s_obj_bce = f32(0.0)
    s_all_bce = f32(0.0)
    n_obj = f32(0.0)
    s_box = f32(0.0)
    s_cls = f32(0.0)
    n_tgt = f32(0.0)

    for a in range(A):
        aw, ah = anchors_grid[a]
        px = p[4 * a + 0]
        py = p[4 * a + 1]
        pw = p[4 * a + 2]
        ph = p[4 * a + 3]
        obj = p[4 * A + a]
        cls = p[5 * A + a * C:5 * A + (a + 1) * C]

        tx = (t0_ref[0, a] * w - x_off) * (1.0 / aw)
        ty = (t1_ref[0, a] * h - y_off) * (1.0 / ah)
        tw = (t2_ref[0, a] * w - x_off) * (1.0 / aw)
        th = (t3_ref[0, a] * h - y_off) * (1.0 / ah)
        tgt_obj = t4_ref[0, a]
        tgt_cls = t5_ref[0, a]

        ax1 = px - pw * 0.5
        ax2 = px + pw * 0.5
        ay1 = py - ph * 0.5
        ay2 = py + ph * 0.5
        bx1 = tx - tw * 0.5
        bx2 = tx + tw * 0.5
        by1 = ty - th * 0.5
        by2 = ty + th * 0.5
        iw = jnp.clip(jnp.minimum(ax2, bx2) - jnp.maximum(ax1, bx1), 0.0)
        ih = jnp.clip(jnp.minimum(ay2, by2) - jnp.maximum(ay1, by1), 0.0)
        inter = iw * ih
        area_a = jnp.clip(ax2 - ax1, 0.0) * jnp.clip(ay2 - ay1, 0.0)
        area_b = jnp.clip(bx2 - bx1, 0.0) * jnp.clip(by2 - by1, 0.0)
        iou = inter / (area_a + area_b - inter + 1e-09)

        tgt_mask = tgt_obj > 0.0
        obj_mask = jnp.logical_and(iou > _IGNORE_THRESH, tgt_mask)
        m = obj_mask.astype(f32)

        obj_bce = jnp.maximum(obj, 0.0) - obj * tgt_obj + _softplus_neg_abs(obj)
        s_all_bce += jnp.sum(obj_bce)
        s_obj_bce += jnp.sum(obj_bce * m)
        n_obj += jnp.sum(m)
        n_tgt += jnp.sum(tgt_mask.astype(f32))

        box_mse = ((px - tx) ** 2 + (py - ty) ** 2
                   + (pw - tw) ** 2 + (ph - th) ** 2) * 0.25
        s_box += jnp.sum(box_mse * m)

        sp = jnp.maximum(cls, 0.0) + _softplus_neg_abs(cls)
        q = sp - jnp.where(cidx == tgt_cls[None].astype(jnp.int32), cls, 0.0)
        cls_bce = jnp.sum(q, axis=0) * (1.0 / C)
        s_cls += jnp.sum(cls_bce * m)

    acc_ref[0] += s_obj_bce
    acc_ref[1] += s_all_bce
    acc_ref[2] += n_obj
    acc_ref[3] += s_box
    acc_ref[4] += s_cls
    acc_ref[5] += n_tgt

    @pl.when(pl.program_id(0) == nb - 1)
    def _finalize():
        so = acc_ref[0]
        sa = acc_ref[1]
        no = acc_ref[2]
        sb = acc_ref[3]
        sc = acc_ref[4]
        nt = acc_ref[5]
        total = f32(nb * A * h * w)
        n_noobj = total - no
        s_noobj = sa - so
        obj_loss = (total / (no + _EPS)) * (so / (no + _EPS))
        noobj_loss = (total / (n_noobj + _EPS)) * (s_noobj / (n_noobj + _EPS))
        box_loss = sb / (no + _EPS)
        class_loss = (total / (nt + _EPS)) * (sc / (no + _EPS))
        out_ref[0, 0] = obj_loss + noobj_loss + box_loss + class_loss


def kernel(predictions, targets):
    b, ch, h, w = predictions.shape
    A = len(_ANCHORS)
    stride = _IMG_SIZE / h
    anchors_grid = tuple((aw / stride, ah / stride) for aw, ah in _ANCHORS)
    R, L = 32, 128
    assert h * w == R * L

    pred = predictions.reshape(b, ch, R, L)
    tplanes = [targets[..., j].reshape(b, A, R, L) for j in range(6)]

    plane_spec = pl.BlockSpec((1, A, R, L), lambda i: (i, 0, 0, 0))
    loss = pl.pallas_call(
        functools.partial(_loss_kernel, h=h, w=w, nb=b,
                          anchors_grid=anchors_grid),
        grid=(b,),
        in_specs=[pl.BlockSpec((1, ch, R, L), lambda i: (i, 0, 0, 0))]
        + [plane_spec] * 6,
        out_specs=pl.BlockSpec(memory_space=pltpu.SMEM),
        out_shape=jax.ShapeDtypeStruct((1, 1), jnp.float32),
        scratch_shapes=[pltpu.SMEM((6,), jnp.float32)],
    )(pred, *tplanes)

    return loss[0, 0]
